# Initial kernel scaffold; baseline (speedup 1.0000x reference)
import jax
import jax.numpy as jnp
from jax.experimental import pallas as pl

BLOCK_M = 1024


def _vq_kernel(x_ref, embed_ref, out_ref):
    x = x_ref[...]
    embed = embed_ref[...]
    dots = jax.lax.dot_general(
        x, embed,
        dimension_numbers=(((1,), (1,)), ((), ())),
        preferred_element_type=jnp.float32,
    )
    norms = jnp.sum(embed * embed, axis=1)[None, :]
    dist = norms - 2.0 * dots
    idx = jnp.argmin(dist, axis=1)
    k_iota = jax.lax.broadcasted_iota(jnp.int32, dist.shape, 1)
    onehot = (k_iota == idx[:, None].astype(jnp.int32)).astype(jnp.float32)
    out_ref[...] = jax.lax.dot_general(
        onehot, embed,
        dimension_numbers=(((1,), (0,)), ((), ())),
        preferred_element_type=jnp.float32,
    )


def kernel(x, embed):
    shape = x.shape
    d = shape[-1]
    flat = x.reshape(-1, d)
    n = flat.shape[0]
    grid = (n // BLOCK_M,)
    quant = pl.pallas_call(
        _vq_kernel,
        grid=grid,
        in_specs=[
            pl.BlockSpec((BLOCK_M, d), lambda i: (i, 0)),
            pl.BlockSpec(embed.shape, lambda i: (0, 0)),
        ],
        out_specs=pl.BlockSpec((BLOCK_M, d), lambda i: (i, 0)),
        out_shape=jax.ShapeDtypeStruct((n, d), jnp.float32),
    )(flat, embed)
    return (quant.reshape(shape), 0)


# speedup vs baseline: 1.2626x; 1.2626x over previous
import jax
import jax.numpy as jnp
from jax.experimental import pallas as pl

BLOCK_M = 256


def _vq_kernel(x_ref, embed_t_ref, embed_ref, out_ref):
    x = x_ref[...]
    embed_t = embed_t_ref[...]
    embed = embed_ref[...]
    dots = jax.lax.dot_general(
        x, embed_t,
        dimension_numbers=(((1,), (0,)), ((), ())),
        preferred_element_type=jnp.float32,
    )
    norms = jnp.sum(embed_t * embed_t, axis=0, keepdims=True)
    dist = norms - 2.0 * dots
    k = dist.shape[1]
    mdist = jnp.min(dist, axis=1, keepdims=True)
    k_iota = jax.lax.broadcasted_iota(jnp.int32, dist.shape, 1)
    masked = jnp.where(dist == mdist, k_iota, k)
    idx = jnp.min(masked, axis=1, keepdims=True)
    onehot = (k_iota == idx).astype(jnp.float32)
    out_ref[...] = jax.lax.dot_general(
        onehot, embed,
        dimension_numbers=(((1,), (0,)), ((), ())),
        preferred_element_type=jnp.float32,
    )


def kernel(x, embed):
    shape = x.shape
    d = shape[-1]
    flat = x.reshape(-1, d)
    n = flat.shape[0]
    embed_t = embed.T
    grid = (n // BLOCK_M,)
    quant = pl.pallas_call(
        _vq_kernel,
        grid=grid,
        in_specs=[
            pl.BlockSpec((BLOCK_M, d), lambda i: (i, 0)),
            pl.BlockSpec(embed_t.shape, lambda i: (0, 0)),
            pl.BlockSpec(embed.shape, lambda i: (0, 0)),
        ],
        out_specs=pl.BlockSpec((BLOCK_M, d), lambda i: (i, 0)),
        out_shape=jax.ShapeDtypeStruct((n, d), jnp.float32),
    )(flat, embed_t, embed)
    return (quant.reshape(shape), 0)


# speedup vs baseline: 2.0384x; 1.6144x over previous
import jax
import jax.numpy as jnp
from jax.experimental import pallas as pl

BLOCK_M = 2048


def _vq_kernel(x_ref, embed_t_ref, embed_ref, out_ref):
    x = x_ref[...]
    embed_t = embed_t_ref[...]
    embed = embed_ref[...]
    dots = jax.lax.dot_general(
        x, embed_t,
        dimension_numbers=(((1,), (0,)), ((), ())),
        preferred_element_type=jnp.float32,
    )
    norms = jnp.sum(embed_t * embed_t, axis=0, keepdims=True)
    dist = norms - 2.0 * dots
    k = dist.shape[1]
    mdist = jnp.min(dist, axis=1, keepdims=True)
    k_iota = jax.lax.broadcasted_iota(jnp.int32, dist.shape, 1).astype(jnp.float32)
    masked = jnp.where(dist == mdist, k_iota, float(k))
    idx = jnp.min(masked, axis=1, keepdims=True)
    onehot = (k_iota == idx).astype(jnp.float32)
    out_ref[...] = jax.lax.dot_general(
        onehot, embed,
        dimension_numbers=(((1,), (0,)), ((), ())),
        preferred_element_type=jnp.float32,
    )


def kernel(x, embed):
    shape = x.shape
    d = shape[-1]
    flat = x.reshape(-1, d)
    n = flat.shape[0]
    embed_t = embed.T
    grid = (n // BLOCK_M,)
    quant = pl.pallas_call(
        _vq_kernel,
        grid=grid,
        in_specs=[
            pl.BlockSpec((BLOCK_M, d), lambda i: (i, 0)),
            pl.BlockSpec(embed_t.shape, lambda i: (0, 0)),
            pl.BlockSpec(embed.shape, lambda i: (0, 0)),
        ],
        out_specs=pl.BlockSpec((BLOCK_M, d), lambda i: (i, 0)),
        out_shape=jax.ShapeDtypeStruct((n, d), jnp.float32),
    )(flat, embed_t, embed)
    return (quant.reshape(shape), 0)
